# baseline (speedup 1.0000x reference)
import functools

import jax
import jax.numpy as jnp
from jax import lax
from jax.experimental import pallas as pl
from jax.experimental.pallas import tpu as pltpu
from jax.experimental.pallas import tpu_sc as plsc

N_NODES = 10000
NP = 10240
E = 320000
NW = 16
EW = E // NW
NPW = NP // NW
L = 16


BN = 2048
BE = 65536
E_PAD = BE * (NP // BN)


def _proj_body(cp_ref, x_ref, ei_ref, z_ref, c_ref, pk_ref):
    od = jax.lax.dot_general(
        cp_ref[...], x_ref[...], (((1,), (1,)), ((), ())),
        preferred_element_type=jnp.float32)
    z_ref[...] = od[0]
    c_ref[...] = od[1]
    ei = ei_ref[...]
    pk_ref[...] = jnp.bitwise_or(jnp.left_shift(ei[1], 16), ei[0])


def _project(x, cp, edge_index):
    return pl.pallas_call(
        _proj_body,
        grid=(NP // BN,),
        in_specs=[
            pl.BlockSpec((8, 128), lambda j: (0, 0)),
            pl.BlockSpec((BN, 128), lambda j: (j, 0)),
            pl.BlockSpec((2, BE), lambda j: (0, j)),
        ],
        out_specs=[
            pl.BlockSpec((BN,), lambda j: (j,)),
            pl.BlockSpec((BN,), lambda j: (j,)),
            pl.BlockSpec((BE,), lambda j: (j,)),
        ],
        out_shape=[
            jax.ShapeDtypeStruct((NP,), jnp.float32),
            jax.ShapeDtypeStruct((NP,), jnp.float32),
            jax.ShapeDtypeStruct((E_PAD,), jnp.int32),
        ],
    )(cp, x, edge_index)


def _sc_gnn(pk, z0, c0, prm):
    mesh = plsc.VectorSubcoreMesh(core_axis_name="c", subcore_axis_name="s",
                                  num_cores=1)

    @functools.partial(
        pl.kernel,
        out_type=[
            jax.ShapeDtypeStruct((NW, L), jnp.float32),
            jax.ShapeDtypeStruct((NW, NP), jnp.float32),
            jax.ShapeDtypeStruct((NP,), jnp.float32),
        ],
        mesh=mesh,
        compiler_params=pltpu.CompilerParams(needs_layout_passes=False),
        scratch_types=[
            pltpu.VMEM((EW,), jnp.int32),
            pltpu.VMEM((NP,), jnp.float32),
            pltpu.VMEM((NP,), jnp.float32),
            pltpu.VMEM((NPW,), jnp.float32),
            pltpu.VMEM((NPW,), jnp.float32),
            pltpu.VMEM((NW, NPW), jnp.float32),
            pltpu.VMEM((8, L), jnp.float32),
            pltpu.VMEM((L,), jnp.float32),
            pltpu.SemaphoreType.DMA,
        ],
    )
    def body(pk_hbm, z0_hbm, c0_hbm, prm_hbm, out_hbm, part_sh, g_sh,
             pk_v, g_tab, agg, c_v, gstage, red, prm_v, accst, sem):
        w = lax.axis_index("s")
        ebase = w * EW
        nbase = w * NPW

        stage = [
            pltpu.async_copy(pk_hbm.at[pl.ds(ebase, EW)], pk_v, sem),
            pltpu.async_copy(prm_hbm, prm_v, sem),
            pltpu.async_copy(z0_hbm, g_tab, sem),
            pltpu.async_copy(c0_hbm.at[pl.ds(nbase, NPW)], c_v, sem),
        ]
        for cp_ in stage:
            cp_.wait()

        lane_iota = lax.iota(jnp.int32, L)

        b0v = prm_v[6]

        @plsc.parallel_loop(0, NPW, step=L, unroll=4)
        def _(j):
            off = pl.multiple_of(j, L)
            c_v[pl.ds(off, L)] = c_v[pl.ds(off, L)] + b0v

        acc = jnp.zeros((L,), jnp.float32)
        pending_g = None
        for layer in range(3):
            @plsc.parallel_loop(0, NP, step=L, unroll=8)
            def _(i):
                agg[pl.ds(pl.multiple_of(i, L), L)] = jnp.zeros(
                    (L,), jnp.float32)

            if pending_g is not None:
                pending_g.wait()
                pending_g = None

            @plsc.parallel_loop(0, EW, step=L, unroll=8)
            def _(i):
                off = pl.multiple_of(i, L)
                p = pk_v[pl.ds(off, L)]
                s = jnp.bitwise_and(p, 0xFFFF)
                d = jnp.right_shift(p, 16)
                v = plsc.load_gather(g_tab, [s])
                plsc.addupdate_scatter(agg, [d], v)

            pltpu.sync_copy(agg, part_sh.at[w])
            plsc.subcore_barrier()
            reds = [
                pltpu.async_copy(part_sh.at[t, pl.ds(nbase, NPW)],
                                 red.at[t], sem)
                for t in range(NW)
            ]
            for cp_ in reds:
                cp_.wait()

            Wv = prm_v[3 * layer + 0] if layer < 2 else None
            Sv = prm_v[3 * layer + 1] if layer < 2 else None
            bv = prm_v[3 * layer + 2] if layer < 2 else None

            if layer < 2:
                @plsc.parallel_loop(0, NPW, step=L, unroll=2)
                def _(j):
                    off = pl.multiple_of(j, L)
                    sv = red[0, pl.ds(off, L)]
                    for t in range(1, NW):
                        sv = sv + red[t, pl.ds(off, L)]
                    hv = jnp.maximum(sv + c_v[pl.ds(off, L)], 0.0)
                    gstage[pl.ds(off, L)] = Wv * hv
                    c_v[pl.ds(off, L)] = Sv * hv + bv

                pltpu.sync_copy(gstage, g_sh.at[pl.ds(nbase, NPW)])
                plsc.subcore_barrier()
                pending_g = pltpu.async_copy(g_sh, g_tab, sem)
            else:
                @plsc.parallel_loop(0, NPW, step=L, unroll=2, carry=acc)
                def nacc(j, carry):
                    off = pl.multiple_of(j, L)
                    sv = red[0, pl.ds(off, L)]
                    for t in range(1, NW):
                        sv = sv + red[t, pl.ds(off, L)]
                    hv = jnp.maximum(sv + c_v[pl.ds(off, L)], 0.0)
                    gidx = lane_iota + (nbase + off)
                    return carry + jnp.where(gidx < N_NODES, hv, 0.0)
                acc = nacc

        accst[...] = acc
        pltpu.sync_copy(accst, out_hbm.at[w])

    return body(pk, z0, c0, prm)[0]


def _readout_body(sums_ref, f1w_ref, f1b_ref, ow_ref, ob_ref, o_ref):
    hg = jnp.sum(sums_ref[...])
    t = (hg * f1w_ref[...] + f1b_ref[...]) * 1000.0
    sg = 1.0 / (1.0 + jnp.exp(-t))
    o = jnp.dot(sg, ow_ref[...],
                preferred_element_type=jnp.float32) + ob_ref[...]
    o = jnp.maximum(o, 0.0)
    m = jnp.max(o, axis=1, keepdims=True)
    lse = jnp.log(jnp.sum(jnp.exp(o - m), axis=1, keepdims=True)) + m
    o_ref[...] = o - lse


def _readout(sums, f1w, f1b, ow, ob):
    return pl.pallas_call(
        _readout_body,
        out_shape=jax.ShapeDtypeStruct((1, 4), jnp.float32),
    )(sums, f1w, f1b, ow, ob)


def kernel(x, edge_index, W0, b0, S0, W1, b1, S1, W2, b2, S2,
           fc1_W, fc1_b, out_W, out_b):
    cp = jnp.zeros((8, 128), jnp.float32)
    cp = cp.at[0].set(W0[:, 0]).at[1].set(S0[0])
    z0, c0, pk = _project(x, cp, edge_index)

    prm = jnp.zeros((8, L), jnp.float32)
    for i, val in enumerate([W1[0, 0], S1[0, 0], b1[0],
                             W2[0, 0], S2[0, 0], b2[0], b0[0]]):
        prm = prm.at[i].set(val)

    sums = _sc_gnn(pk, z0, c0, prm)

    return _readout(sums, fc1_W.T, fc1_b.reshape(1, 8),
                    out_W.T, out_b.reshape(1, 4))


# speedup vs baseline: 1.0447x; 1.0447x over previous
import functools

import jax
import jax.numpy as jnp
from jax import lax
from jax.experimental import pallas as pl
from jax.experimental.pallas import tpu as pltpu
from jax.experimental.pallas import tpu_sc as plsc

N_NODES = 10000
NP = 10240
E = 320000
NW = 16
EW = E // NW
NPW = NP // NW
L = 16


BN = 1024
BE = 32768
E_PAD = BE * (NP // BN)


def _proj_body(cp_ref, x_ref, ei_ref, z_ref, c_ref, pk_ref):
    od = jax.lax.dot_general(
        cp_ref[...], x_ref[...], (((1,), (1,)), ((), ())),
        preferred_element_type=jnp.float32)
    z_ref[...] = od[0]
    c_ref[...] = od[1]
    ei = ei_ref[...]
    pk_ref[...] = jnp.bitwise_or(jnp.left_shift(ei[1], 16), ei[0])


def _project(x, cp, edge_index):
    return pl.pallas_call(
        _proj_body,
        grid=(NP // BN,),
        in_specs=[
            pl.BlockSpec((8, 128), lambda j: (0, 0)),
            pl.BlockSpec((BN, 128), lambda j: (j, 0)),
            pl.BlockSpec((2, BE), lambda j: (0, j)),
        ],
        out_specs=[
            pl.BlockSpec((BN,), lambda j: (j,)),
            pl.BlockSpec((BN,), lambda j: (j,)),
            pl.BlockSpec((BE,), lambda j: (j,)),
        ],
        out_shape=[
            jax.ShapeDtypeStruct((NP,), jnp.float32),
            jax.ShapeDtypeStruct((NP,), jnp.float32),
            jax.ShapeDtypeStruct((E_PAD,), jnp.int32),
        ],
    )(cp, x, edge_index)


def _sc_gnn(pk, z0, c0, prm):
    mesh = plsc.VectorSubcoreMesh(core_axis_name="c", subcore_axis_name="s",
                                  num_cores=1)

    @functools.partial(
        pl.kernel,
        out_type=jax.ShapeDtypeStruct((NW, L), jnp.float32),
        mesh=mesh,
        compiler_params=pltpu.CompilerParams(needs_layout_passes=False),
        scratch_types=[
            pltpu.VMEM((EW,), jnp.int32),
            pltpu.VMEM((NP,), jnp.float32),
            pltpu.VMEM((NP,), jnp.float32),
            pltpu.VMEM((NPW,), jnp.float32),
            pltpu.VMEM((NPW,), jnp.float32),
            pltpu.VMEM((NW, NPW), jnp.float32),
            pltpu.VMEM((8, L), jnp.float32),
            pltpu.VMEM((L,), jnp.float32),
            pltpu.VMEM_SHARED((NW, NP), jnp.float32),
            pltpu.VMEM_SHARED((NP,), jnp.float32),
            pltpu.SemaphoreType.DMA,
        ],
    )
    def body(pk_hbm, z0_hbm, c0_hbm, prm_hbm, out_hbm, pk_v, g_tab,
             agg, c_v, gstage, red, prm_v, accst, part_sh, g_sh, sem):
        w = lax.axis_index("s")
        ebase = w * EW
        nbase = w * NPW

        stage = [
            pltpu.async_copy(pk_hbm.at[pl.ds(ebase, EW)], pk_v, sem),
            pltpu.async_copy(prm_hbm, prm_v, sem),
            pltpu.async_copy(z0_hbm, g_tab, sem),
            pltpu.async_copy(c0_hbm.at[pl.ds(nbase, NPW)], c_v, sem),
        ]
        for cp_ in stage:
            cp_.wait()

        lane_iota = lax.iota(jnp.int32, L)

        b0v = prm_v[6]

        @plsc.parallel_loop(0, NPW, step=L, unroll=4)
        def _(j):
            off = pl.multiple_of(j, L)
            c_v[pl.ds(off, L)] = c_v[pl.ds(off, L)] + b0v

        acc = jnp.zeros((L,), jnp.float32)
        pending_g = None
        for layer in range(3):
            @plsc.parallel_loop(0, NP, step=L, unroll=8)
            def _(i):
                agg[pl.ds(pl.multiple_of(i, L), L)] = jnp.zeros(
                    (L,), jnp.float32)

            if pending_g is not None:
                pending_g.wait()
                pending_g = None

            @plsc.parallel_loop(0, EW, step=L, unroll=8)
            def _(i):
                off = pl.multiple_of(i, L)
                p = pk_v[pl.ds(off, L)]
                s = jnp.bitwise_and(p, 0xFFFF)
                d = jnp.right_shift(p, 16)
                v = plsc.load_gather(g_tab, [s])
                plsc.addupdate_scatter(agg, [d], v)

            pltpu.sync_copy(agg, part_sh.at[w])
            plsc.subcore_barrier()
            reds = [
                pltpu.async_copy(part_sh.at[t, pl.ds(nbase, NPW)],
                                 red.at[t], sem)
                for t in range(NW)
            ]
            for cp_ in reds:
                cp_.wait()

            Wv = prm_v[3 * layer + 0] if layer < 2 else None
            Sv = prm_v[3 * layer + 1] if layer < 2 else None
            bv = prm_v[3 * layer + 2] if layer < 2 else None

            if layer < 2:
                @plsc.parallel_loop(0, NPW, step=L, unroll=2)
                def _(j):
                    off = pl.multiple_of(j, L)
                    sv = red[0, pl.ds(off, L)]
                    for t in range(1, NW):
                        sv = sv + red[t, pl.ds(off, L)]
                    hv = jnp.maximum(sv + c_v[pl.ds(off, L)], 0.0)
                    gstage[pl.ds(off, L)] = Wv * hv
                    c_v[pl.ds(off, L)] = Sv * hv + bv

                pltpu.sync_copy(gstage, g_sh.at[pl.ds(nbase, NPW)])
                plsc.subcore_barrier()
                pending_g = pltpu.async_copy(g_sh, g_tab, sem)
            else:
                @plsc.parallel_loop(0, NPW, step=L, unroll=2, carry=acc)
                def nacc(j, carry):
                    off = pl.multiple_of(j, L)
                    sv = red[0, pl.ds(off, L)]
                    for t in range(1, NW):
                        sv = sv + red[t, pl.ds(off, L)]
                    hv = jnp.maximum(sv + c_v[pl.ds(off, L)], 0.0)
                    gidx = lane_iota + (nbase + off)
                    return carry + jnp.where(gidx < N_NODES, hv, 0.0)
                acc = nacc

        accst[...] = acc
        pltpu.sync_copy(accst, out_hbm.at[w])

    return body(pk, z0, c0, prm)


def _readout_body(sums_ref, f1w_ref, f1b_ref, ow_ref, ob_ref, o_ref):
    hg = jnp.sum(sums_ref[...])
    t = (hg * f1w_ref[...] + f1b_ref[...]) * 1000.0
    sg = 1.0 / (1.0 + jnp.exp(-t))
    o = jnp.sum(ow_ref[...] * sg, axis=1).reshape(1, 4) + ob_ref[...]
    o = jnp.maximum(o, 0.0)
    m = jnp.max(o, axis=1, keepdims=True)
    lse = jnp.log(jnp.sum(jnp.exp(o - m), axis=1, keepdims=True)) + m
    o_ref[...] = o - lse


def _readout(sums, f1w, f1b, ow, ob):
    return pl.pallas_call(
        _readout_body,
        out_shape=jax.ShapeDtypeStruct((1, 4), jnp.float32),
    )(sums, f1w, f1b, ow, ob)


def kernel(x, edge_index, W0, b0, S0, W1, b1, S1, W2, b2, S2,
           fc1_W, fc1_b, out_W, out_b):
    cp = jnp.zeros((8, 128), jnp.float32)
    cp = cp.at[0].set(W0[:, 0]).at[1].set(S0[0])
    z0, c0, pk = _project(x, cp, edge_index)

    prm = jnp.zeros((8, L), jnp.float32)
    for i, val in enumerate([W1[0, 0], S1[0, 0], b1[0],
                             W2[0, 0], S2[0, 0], b2[0], b0[0]]):
        prm = prm.at[i].set(val)

    sums = _sc_gnn(pk, z0, c0, prm)

    return _readout(sums, fc1_W.T, fc1_b.reshape(1, 8),
                    out_W, out_b.reshape(1, 4))


# speedup vs baseline: 1.1047x; 1.0575x over previous
import functools

import jax
import jax.numpy as jnp
from jax import lax
from jax.experimental import pallas as pl
from jax.experimental.pallas import tpu as pltpu
from jax.experimental.pallas import tpu_sc as plsc

N_NODES = 10000
NP = 10240
E = 320000
NW = 16
EW = E // NW
NPW = NP // NW
L = 16


BN = 2048
BE = 65536
E_PAD = BE * (NP // BN)


def _proj_body(cp_ref, x_ref, ei_ref, z_ref, c_ref, pk_ref):
    od = jax.lax.dot_general(
        cp_ref[...], x_ref[...], (((1,), (1,)), ((), ())),
        preferred_element_type=jnp.float32)
    z_ref[...] = od[0]
    c_ref[...] = od[1]
    ei = ei_ref[...]
    pk_ref[...] = jnp.bitwise_or(jnp.left_shift(ei[1], 16), ei[0])


def _project(x, cp, edge_index):
    return pl.pallas_call(
        _proj_body,
        grid=(NP // BN,),
        in_specs=[
            pl.BlockSpec((8, 128), lambda j: (0, 0)),
            pl.BlockSpec((BN, 128), lambda j: (j, 0)),
            pl.BlockSpec((2, BE), lambda j: (0, j)),
        ],
        out_specs=[
            pl.BlockSpec((BN,), lambda j: (j,)),
            pl.BlockSpec((BN,), lambda j: (j,)),
            pl.BlockSpec((BE,), lambda j: (j,)),
        ],
        out_shape=[
            jax.ShapeDtypeStruct((NP,), jnp.float32),
            jax.ShapeDtypeStruct((NP,), jnp.float32),
            jax.ShapeDtypeStruct((E_PAD,), jnp.int32),
        ],
    )(cp, x, edge_index)


def _sc_gnn(pk, z0, c0, prm):
    mesh = plsc.VectorSubcoreMesh(core_axis_name="c", subcore_axis_name="s",
                                  num_cores=1)

    @functools.partial(
        pl.kernel,
        out_type=jax.ShapeDtypeStruct((NW, L), jnp.float32),
        mesh=mesh,
        compiler_params=pltpu.CompilerParams(needs_layout_passes=False),
        scratch_types=[
            pltpu.VMEM((EW,), jnp.int32),
            pltpu.VMEM((NP,), jnp.float32),
            pltpu.VMEM((NP,), jnp.float32),
            pltpu.VMEM((NPW,), jnp.float32),
            pltpu.VMEM((NPW,), jnp.float32),
            pltpu.VMEM((NW, NPW), jnp.float32),
            pltpu.VMEM((8, L), jnp.float32),
            pltpu.VMEM((L,), jnp.float32),
            pltpu.VMEM_SHARED((NW, NW, NPW), jnp.float32),
            pltpu.VMEM_SHARED((NP,), jnp.float32),
            pltpu.SemaphoreType.DMA,
        ],
    )
    def body(pk_hbm, z0_hbm, c0_hbm, prm_hbm, out_hbm, pk_v, g_tab,
             agg, c_v, gstage, red, prm_v, accst, part_sh, g_sh, sem):
        w = lax.axis_index("s")
        ebase = w * EW
        nbase = w * NPW

        stage = [
            pltpu.async_copy(pk_hbm.at[pl.ds(ebase, EW)], pk_v, sem),
            pltpu.async_copy(prm_hbm, prm_v, sem),
            pltpu.async_copy(z0_hbm, g_tab, sem),
            pltpu.async_copy(c0_hbm.at[pl.ds(nbase, NPW)], c_v, sem),
        ]
        for cp_ in stage:
            cp_.wait()

        lane_iota = lax.iota(jnp.int32, L)

        b0v = prm_v[6]

        @plsc.parallel_loop(0, NPW, step=L, unroll=4)
        def _(j):
            off = pl.multiple_of(j, L)
            c_v[pl.ds(off, L)] = c_v[pl.ds(off, L)] + b0v

        acc = jnp.zeros((L,), jnp.float32)
        pending_g = None
        for layer in range(3):
            @plsc.parallel_loop(0, NP, step=L, unroll=8)
            def _(i):
                agg[pl.ds(pl.multiple_of(i, L), L)] = jnp.zeros(
                    (L,), jnp.float32)

            if pending_g is not None:
                pending_g.wait()
                pending_g = None

            @plsc.parallel_loop(0, EW, step=L, unroll=8)
            def _(i):
                off = pl.multiple_of(i, L)
                p = pk_v[pl.ds(off, L)]
                s = jnp.bitwise_and(p, 0xFFFF)
                d = jnp.right_shift(p, 16)
                v = plsc.load_gather(g_tab, [s])
                plsc.addupdate_scatter(agg, [d], v)

            pubs = [
                pltpu.async_copy(agg.at[pl.ds(t * NPW, NPW)],
                                 part_sh.at[t, w], sem)
                for t in range(NW)
            ]
            for cp_ in pubs:
                cp_.wait()
            plsc.subcore_barrier()
            pltpu.sync_copy(part_sh.at[w], red)

            Wv = prm_v[3 * layer + 0] if layer < 2 else None
            Sv = prm_v[3 * layer + 1] if layer < 2 else None
            bv = prm_v[3 * layer + 2] if layer < 2 else None

            if layer < 2:
                @plsc.parallel_loop(0, NPW, step=L, unroll=2)
                def _(j):
                    off = pl.multiple_of(j, L)
                    sv = red[0, pl.ds(off, L)]
                    for t in range(1, NW):
                        sv = sv + red[t, pl.ds(off, L)]
                    hv = jnp.maximum(sv + c_v[pl.ds(off, L)], 0.0)
                    gstage[pl.ds(off, L)] = Wv * hv
                    c_v[pl.ds(off, L)] = Sv * hv + bv

                pltpu.sync_copy(gstage, g_sh.at[pl.ds(nbase, NPW)])
                plsc.subcore_barrier()
                pending_g = pltpu.async_copy(g_sh, g_tab, sem)
            else:
                @plsc.parallel_loop(0, NPW, step=L, unroll=2, carry=acc)
                def nacc(j, carry):
                    off = pl.multiple_of(j, L)
                    sv = red[0, pl.ds(off, L)]
                    for t in range(1, NW):
                        sv = sv + red[t, pl.ds(off, L)]
                    hv = jnp.maximum(sv + c_v[pl.ds(off, L)], 0.0)
                    gidx = lane_iota + (nbase + off)
                    return carry + jnp.where(gidx < N_NODES, hv, 0.0)
                acc = nacc

        accst[...] = acc
        pltpu.sync_copy(accst, out_hbm.at[w])

    return body(pk, z0, c0, prm)


def _readout_body(sums_ref, f1w_ref, f1b_ref, ow_ref, ob_ref, o_ref):
    hg = jnp.sum(sums_ref[...])
    t = (hg * f1w_ref[...] + f1b_ref[...]) * 1000.0
    sg = 1.0 / (1.0 + jnp.exp(-t))
    o = jnp.sum(ow_ref[...] * sg, axis=1).reshape(1, 4) + ob_ref[...]
    o = jnp.maximum(o, 0.0)
    m = jnp.max(o, axis=1, keepdims=True)
    lse = jnp.log(jnp.sum(jnp.exp(o - m), axis=1, keepdims=True)) + m
    o_ref[...] = o - lse


def _readout(sums, f1w, f1b, ow, ob):
    return pl.pallas_call(
        _readout_body,
        out_shape=jax.ShapeDtypeStruct((1, 4), jnp.float32),
    )(sums, f1w, f1b, ow, ob)


def kernel(x, edge_index, W0, b0, S0, W1, b1, S1, W2, b2, S2,
           fc1_W, fc1_b, out_W, out_b):
    cp = jnp.zeros((8, 128), jnp.float32)
    cp = cp.at[0].set(W0[:, 0]).at[1].set(S0[0])
    z0, c0, pk = _project(x, cp, edge_index)

    prm = jnp.zeros((8, L), jnp.float32)
    for i, val in enumerate([W1[0, 0], S1[0, 0], b1[0],
                             W2[0, 0], S2[0, 0], b2[0], b0[0]]):
        prm = prm.at[i].set(val)

    sums = _sc_gnn(pk, z0, c0, prm)

    return _readout(sums, fc1_W.T, fc1_b.reshape(1, 8),
                    out_W, out_b.reshape(1, 4))
